# Initial kernel scaffold; baseline (speedup 1.0000x reference)
import jax
import jax.numpy as jnp
from jax.experimental import pallas as pl


def kernel(feat_r1, feat_r2, edge_index_r1, edge_index_r2, rel_emb_r1, rel_emb_r2, W_node, W_rel_r1, W_rel_r2, W_res, b_res, res_w, W_upd_r1, b_upd_r1, W_upd_r2, b_upd_r2, rel_attn_r1, rel_attn_r2):
    raise NotImplementedError("write your pallas kernel here")


# speedup vs baseline: 14.5050x; 14.5050x over previous
import jax
import jax.numpy as jnp
from jax import lax
from jax.experimental import pallas as pl
from jax.experimental.pallas import tpu as pltpu
from jax.experimental.pallas import tpu_sc as plsc

N = 10000
E = 320000
K = 8
D = 16
KD = K * D
W_EXT = 256
NEG = 0.2

NC = 2
NS = 16
EPT = E // NS
C = 80
NCHUNK = EPT // C
HALF = 5120
RPT = HALF // NS
DR = 64
NDR = RPT // DR
DROW = HALF + 8
NDROW = HALF // 8 + 24
ACC_ROWS = DROW + NDROW


def _sel_matrices():
    j = lax.broadcasted_iota(jnp.int32, (KD, K), 0)
    k = lax.broadcasted_iota(jnp.int32, (KD, K), 1)
    S = (j // D == k).astype(jnp.float32)
    k2 = lax.broadcasted_iota(jnp.int32, (K, KD), 0)
    j2 = lax.broadcasted_iota(jnp.int32, (K, KD), 1)
    St = (j2 // D == k2).astype(jnp.float32)
    return S, St


def _attn_perms():
    i = lax.broadcasted_iota(jnp.int32, (2 * KD, KD), 0)
    j = lax.broadcasted_iota(jnp.int32, (2 * KD, KD), 1)
    tgt = (j // D) * (2 * D) + (j % D)
    Pl = (i == tgt).astype(jnp.float32)
    Pr = (i == tgt + D).astype(jnp.float32)
    return Pl, Pr


def _tc1_body(f1_ref, f2_ref, wn_ref, re1_ref, wr1_ref, re2_ref, wr2_ref,
              wu1_ref, bu1_ref, wu2_ref, bu2_ref,
              fs1_ref, er1_ref, fs2_ref, er2_ref, rn1_ref, rn2_ref):
    g1 = jnp.dot(f1_ref[...], wn_ref[...], preferred_element_type=jnp.float32)
    g2 = jnp.dot(f2_ref[...], wn_ref[...], preferred_element_type=jnp.float32)
    Pl, Pr = _attn_perms()
    S, _ = _sel_matrices()
    a1 = jnp.dot(re1_ref[...], wr1_ref[...], preferred_element_type=jnp.float32)
    a2 = jnp.dot(re2_ref[...], wr2_ref[...], preferred_element_type=jnp.float32)
    a1l = jnp.dot(a1, Pl, preferred_element_type=jnp.float32)
    a1r = jnp.dot(a1, Pr, preferred_element_type=jnp.float32)
    a2l = jnp.dot(a2, Pl, preferred_element_type=jnp.float32)
    a2r = jnp.dot(a2, Pr, preferred_element_type=jnp.float32)
    B = g1.shape[0]
    z120 = jnp.zeros((B, W_EXT - KD - K), jnp.float32)
    zer = jnp.zeros((B, KD - K), jnp.float32)
    el1 = jnp.dot(g2 * a1l, S, preferred_element_type=jnp.float32)
    er1 = jnp.dot(g1 * a1r, S, preferred_element_type=jnp.float32)
    fs1_ref[...] = jnp.concatenate([g2, el1, z120], axis=1)
    er1_ref[...] = jnp.concatenate([er1, zer], axis=1)
    el2 = jnp.dot(g1 * a2l, S, preferred_element_type=jnp.float32)
    er2 = jnp.dot(g2 * a2r, S, preferred_element_type=jnp.float32)
    fs2_ref[...] = jnp.concatenate([g1, el2, z120], axis=1)
    er2_ref[...] = jnp.concatenate([er2, zer], axis=1)

    @pl.when(pl.program_id(0) == 0)
    def _():
        rn1_ref[...] = jnp.dot(re1_ref[...], wu1_ref[...],
                               preferred_element_type=jnp.float32) + bu1_ref[...]
        rn2_ref[...] = jnp.dot(re2_ref[...], wu2_ref[...],
                               preferred_element_type=jnp.float32) + bu2_ref[...]


def _tc1(f1, f2, wn, re1, wr1, re2, wr2, wu1, bu1, wu2, bu2):
    B = 1000
    grid = N // B
    full = lambda shape: pl.BlockSpec(shape, lambda i: tuple(0 for _ in shape))
    row = lambda w: pl.BlockSpec((B, w), lambda i: (i, 0))
    return pl.pallas_call(
        _tc1_body,
        grid=(grid,),
        in_specs=[row(KD), row(KD), full((KD, KD)),
                  full((1, 64)), full((64, 256)), full((1, 64)), full((64, 256)),
                  full((64, 64)), full((1, 64)), full((64, 64)), full((1, 64))],
        out_specs=[row(W_EXT), row(KD), row(W_EXT), row(KD),
                   full((1, 64)), full((1, 64))],
        out_shape=[jax.ShapeDtypeStruct((N, W_EXT), jnp.float32),
                   jax.ShapeDtypeStruct((N, KD), jnp.float32),
                   jax.ShapeDtypeStruct((N, W_EXT), jnp.float32),
                   jax.ShapeDtypeStruct((N, KD), jnp.float32),
                   jax.ShapeDtypeStruct((1, 64), jnp.float32),
                   jax.ShapeDtypeStruct((1, 64), jnp.float32)],
    )(f1, f2, wn, re1, wr1, re2, wr2, wu1, bu1, wu2, bu2)


def _sc_body(fs_hbm, er_hbm, src_hbm, dst_hbm, out_hbm, outw_hbm, outd_hbm,
             srcv, dstv, rowv, rowsv, erv, mv, mden, wbuf, dbuf, accsh, sem):
    cid = lax.axis_index("c")
    sid = lax.axis_index("s")
    nbase = cid * HALF
    lanes = lax.iota(jnp.int32, 16)

    def zrow(r, carry):
        for cc in range(KD // 16):
            dbuf[r, pl.ds(cc * 16, 16)] = jnp.zeros((16,), jnp.float32)
        return carry

    lax.fori_loop(0, DR, zrow, 0)

    def zden(i, carry):
        for cc in range(KD // 16):
            mden[i, pl.ds(cc * 16, 16)] = jnp.zeros((16,), jnp.float32)
        return carry

    lax.fori_loop(0, C, zden, 0)

    def init_acc():
        for t in range(NDR):
            pltpu.sync_copy(dbuf, accsh.at[pl.ds(sid * RPT + t * DR, DR)])

        @pl.when(sid == 0)
        def _():
            pltpu.sync_copy(dbuf.at[pl.ds(0, 8)], accsh.at[pl.ds(HALF, 8)])

    def load_indices(c):
        base = pl.multiple_of(sid * EPT + c * C, 8)
        pltpu.sync_copy(src_hbm.at[pl.ds(base, C)], srcv)
        pltpu.sync_copy(dst_hbm.at[pl.ds(base, C)], dstv)

        def rmap(g, rcarry):
            dl = dstv[pl.ds(g * 16, 16)] - nbase
            ok = (dl >= 0) & (dl < HALF)
            rowv[pl.ds(g * 16, 16)] = jnp.where(ok, dl, HALF)
            return rcarry

        lax.fori_loop(0, C // 16, rmap, 0)
        return base

    init_acc()
    plsc.subcore_barrier()

    def chunk_a(c, carry):
        base = load_indices(c)
        pltpu.async_copy(fs_hbm.at[srcv], rowsv, sem).wait()
        pltpu.async_copy(er_hbm.at[dstv], erv, sem).wait()

        def edge(i, ecarry):
            el = rowsv[i, pl.ds(KD, 16)]
            er = erv[i, pl.ds(0, 16)]
            e = el + er
            w = jnp.exp(jnp.where(e >= 0.0, e, e * NEG))
            wbuf[i, pl.ds(0, 16)] = jnp.where(lanes < K, w, 0.0)
            for k in range(K):
                wk = w[k]
                mv[i, pl.ds(k * D, 16)] = rowsv[i, pl.ds(k * D, 16)] * wk
            return ecarry

        lax.fori_loop(0, C, edge, 0)
        pltpu.sync_copy(mv, accsh.at[rowv], add=True)
        pltpu.sync_copy(wbuf, outw_hbm.at[cid, pl.ds(base, C)])
        return carry

    lax.fori_loop(0, NCHUNK, chunk_a, 0)
    plsc.subcore_barrier()
    for t in range(NDR):
        pltpu.sync_copy(accsh.at[pl.ds(sid * RPT + t * DR, DR)], dbuf)
        pltpu.sync_copy(dbuf, out_hbm.at[cid, pl.ds(sid * RPT + t * DR, DR)])
    plsc.subcore_barrier()

    lax.fori_loop(0, DR, zrow, 0)
    init_acc()
    plsc.subcore_barrier()

    def chunk_b(c, carry):
        base = load_indices(c)
        pltpu.sync_copy(outw_hbm.at[cid, pl.ds(base, C)], wbuf)

        def edge(i, ecarry):
            mden[i, pl.ds(0, 16)] = wbuf[i, pl.ds(0, 16)]
            return ecarry

        lax.fori_loop(0, C, edge, 0)
        pltpu.sync_copy(mden, accsh.at[rowv], add=True)
        return carry

    lax.fori_loop(0, NCHUNK, chunk_b, 0)
    plsc.subcore_barrier()
    for t in range(NDR):
        pltpu.sync_copy(accsh.at[pl.ds(sid * RPT + t * DR, DR)], dbuf)
        pltpu.sync_copy(dbuf, outd_hbm.at[cid, pl.ds(sid * RPT + t * DR, DR)])


_sc_edge_pass = pl.kernel(
    _sc_body,
    mesh=plsc.VectorSubcoreMesh(core_axis_name="c", subcore_axis_name="s"),
    out_type=[jax.ShapeDtypeStruct((NC, HALF, KD), jnp.float32),
              jax.ShapeDtypeStruct((NC, E, 16), jnp.float32),
              jax.ShapeDtypeStruct((NC, HALF, KD), jnp.float32)],
    scratch_types=[
        pltpu.VMEM((C,), jnp.int32),
        pltpu.VMEM((C,), jnp.int32),
        pltpu.VMEM((C,), jnp.int32),
        pltpu.VMEM((C, W_EXT), jnp.float32),
        pltpu.VMEM((C, KD), jnp.float32),
        pltpu.VMEM((C, KD), jnp.float32),
        pltpu.VMEM((C, KD), jnp.float32),
        pltpu.VMEM((C, 16), jnp.float32),
        pltpu.VMEM((DR, KD), jnp.float32),
        pltpu.VMEM_SHARED((HALF + 8, KD), jnp.float32),
        pltpu.SemaphoreType.DMA,
    ],
)


def _tc2_body(x1_ref, x2_ref, d1_ref, d2_ref, f1_ref, f2_ref, wres_ref,
              bres_ref, resw_ref, ra1_ref, ra2_ref, out1_ref, out2_ref):
    S, St = _sel_matrices()
    alpha = jax.nn.sigmoid(resw_ref[0, 0])

    def finish(x_ref, d_ref, f_ref):
        num = x_ref[...]
        den = d_ref[...]
        dfull = jnp.dot(den, St, preferred_element_type=jnp.float32)
        h = jnp.maximum(num / (dfull + 1e-16), 0.0)
        res = jnp.dot(f_ref[...], wres_ref[...],
                      preferred_element_type=jnp.float32) + bres_ref[...]
        return h * alpha + res * (1.0 - alpha)

    h1 = finish(x1_ref, d1_ref, f1_ref)
    h2 = finish(x2_ref, d2_ref, f2_ref)

    def crossing(ra_ref):
        ra = ra_ref[...]
        l1 = jnp.dot(h1 * ra, S, preferred_element_type=jnp.float32)
        l2 = jnp.dot(h2 * ra, S, preferred_element_type=jnp.float32)
        l1 = jnp.where(l1 >= 0.0, l1, l1 * NEG)
        l2 = jnp.where(l2 >= 0.0, l2, l2 * NEG)
        e1 = jnp.exp(l1)
        e2 = jnp.exp(l2)
        s = e1 + e2
        w1 = jnp.dot(e1 / s, St, preferred_element_type=jnp.float32)
        w2 = jnp.dot(e2 / s, St, preferred_element_type=jnp.float32)
        return h1 * w1 + h2 * w2

    out1_ref[...] = crossing(ra1_ref)
    out2_ref[...] = crossing(ra2_ref)


def _tc2(x1, x2, d1, d2, f1, f2, wres, bres, resw, ra1, ra2):
    B = 512
    npb = HALF // B
    grid = (NC * HALF) // B
    full = lambda shape: pl.BlockSpec(shape, lambda i: tuple(0 for _ in shape))
    row = lambda w: pl.BlockSpec((B, w), lambda i: (i, 0))
    dspec = pl.BlockSpec((B, K), lambda i: (i, 0))
    return pl.pallas_call(
        _tc2_body,
        grid=(grid,),
        in_specs=[row(KD), row(KD), dspec, dspec, row(KD), row(KD),
                  full((KD, KD)), full((1, KD)), full((1, 1)),
                  full((1, KD)), full((1, KD))],
        out_specs=[row(KD), row(KD)],
        out_shape=[jax.ShapeDtypeStruct((NC * HALF, KD), jnp.float32),
                   jax.ShapeDtypeStruct((NC * HALF, KD), jnp.float32)],
    )(x1, x2, d1, d2, f1, f2, wres, bres, resw, ra1, ra2)


def kernel(feat_r1, feat_r2, edge_index_r1, edge_index_r2, rel_emb_r1,
           rel_emb_r2, W_node, W_rel_r1, W_rel_r2, W_res, b_res, res_w,
           W_upd_r1, b_upd_r1, W_upd_r2, b_upd_r2, rel_attn_r1, rel_attn_r2):
    fs1, er1, fs2, er2, rn1, rn2 = _tc1(
        feat_r1, feat_r2, W_node,
        rel_emb_r1.reshape(1, 64), W_rel_r1,
        rel_emb_r2.reshape(1, 64), W_rel_r2,
        W_upd_r1, b_upd_r1.reshape(1, 64),
        W_upd_r2, b_upd_r2.reshape(1, 64))
    acc1, _, dac1 = _sc_edge_pass(fs1, er1, edge_index_r1[0], edge_index_r1[1])
    acc2, _, dac2 = _sc_edge_pass(fs2, er2, edge_index_r2[0], edge_index_r2[1])

    def unpack(acc, dac):
        x = jnp.concatenate([acc[0, :HALF], acc[1, :HALF]], axis=0)
        d = jnp.concatenate([dac[0, :HALF, :K], dac[1, :HALF, :K]], axis=0)
        return x, d

    x1, d1 = unpack(acc1, dac1)
    x2, d2 = unpack(acc2, dac2)
    zf = jnp.zeros((NC * HALF - N, KD), jnp.float32)
    f1p = jnp.concatenate([feat_r1, zf], axis=0)
    f2p = jnp.concatenate([feat_r2, zf], axis=0)
    out1, out2 = _tc2(x1, x2, d1, d2, f1p, f2p, W_res,
                      b_res.reshape(1, KD), res_w.reshape(1, 1),
                      rel_attn_r1.reshape(1, KD), rel_attn_r2.reshape(1, KD))
    return out1[:N], out2[:N], rn1.reshape(64), rn2.reshape(64)
